# baseline (speedup 1.0000x reference)
import jax
import jax.numpy as jnp
from jax import lax
from jax.experimental import pallas as pl
from jax.experimental.pallas import tpu as pltpu
from jax.experimental.pallas import tpu_sc as plsc

_W = 2048
_OUT_ROWS = 8 * 2048
_TC_ROWS = 9216
_SC_ROWS = _OUT_ROWS - _TC_ROWS
_TC_BLK = 1024

_NC, _NS = 2, 16
_NW = _NC * _NS
_RPW = _SC_ROWS // _NW
_CHUNK = 16
_NBUF = 3
_NCHUNKS = _RPW // _CHUNK


def _tc_body(x_ref, o_ref):
    o_ref[...] = x_ref[...]


def _tc_copy(xf):
    return pl.pallas_call(
        _tc_body,
        out_shape=jax.ShapeDtypeStruct((_TC_ROWS, _W), jnp.float32),
        grid=(_TC_ROWS // _TC_BLK,),
        in_specs=[pl.BlockSpec((_TC_BLK, _W), lambda i: (i + 6 * (i // 8), 0))],
        out_specs=pl.BlockSpec((_TC_BLK, _W), lambda i: (i, 0)),
    )(xf)


def _sc_body(x_hbm, o_hbm, buf, in_sem, out_sem):
    wid = lax.axis_index("s") * _NC + lax.axis_index("c")
    out_base = wid * _RPW
    src_base = out_base + _TC_ROWS + 6144

    def in_copy(k, slot):
        return pltpu.make_async_copy(
            x_hbm.at[pl.ds(src_base + k * _CHUNK, _CHUNK)],
            buf.at[slot], in_sem.at[slot])

    def out_copy(k, slot):
        return pltpu.make_async_copy(
            buf.at[slot], o_hbm.at[pl.ds(out_base + k * _CHUNK, _CHUNK)],
            out_sem.at[slot])

    for b in range(min(_NBUF, _NCHUNKS)):
        in_copy(b, b).start()
    for k in range(_NCHUNKS):
        slot = k % _NBUF
        in_copy(k, slot).wait()
        out_copy(k, slot).start()
        nk = k + _NBUF
        if nk < _NCHUNKS:
            out_copy(k, slot).wait()
            in_copy(nk, slot).start()
    for k in range(max(0, _NCHUNKS - _NBUF), _NCHUNKS):
        out_copy(k, k % _NBUF).wait()


_sc_copy = pl.kernel(
    _sc_body,
    mesh=plsc.VectorSubcoreMesh(core_axis_name="c", subcore_axis_name="s"),
    out_type=jax.ShapeDtypeStruct((_SC_ROWS, _W), jnp.float32),
    scratch_types=[
        pltpu.VMEM((_NBUF, _CHUNK, _W), jnp.float32),
        pltpu.SemaphoreType.DMA((_NBUF,)),
        pltpu.SemaphoreType.DMA((_NBUF,)),
    ],
)


def kernel(x):
    xf = x.reshape(14 * 2048, _W)
    sc_out = _sc_copy(xf)
    tc_out = _tc_copy(xf)
    return jnp.concatenate([tc_out, sc_out], axis=0).reshape(8, 2048, 2048)


# speedup vs baseline: 1.6640x; 1.6640x over previous
import jax
import jax.numpy as jnp
from jax import lax
from jax.experimental import pallas as pl
from jax.experimental.pallas import tpu as pltpu
from jax.experimental.pallas import tpu_sc as plsc

_W = 2048
_OUT_ROWS = 8 * 2048
_NC, _NS = 2, 16
_NW = _NC * _NS
_RPW = _OUT_ROWS // _NW
_CHUNK = 16
_NBUF = 3
_NCHUNKS = _RPW // _CHUNK


def _sc_body(x_hbm, o_hbm, buf, in_sem, out_sem):
    wid = lax.axis_index("s") * _NC + lax.axis_index("c")
    slab = wid // 4
    src_slab = slab + 3 * (slab // 4)
    within = (wid % 4) * _RPW
    src_base = src_slab * 2048 + within
    out_base = wid * _RPW

    def in_copy(k, slot):
        return pltpu.make_async_copy(
            x_hbm.at[pl.ds(src_base + k * _CHUNK, _CHUNK)],
            buf.at[slot], in_sem.at[slot])

    def out_copy(k, slot):
        return pltpu.make_async_copy(
            buf.at[slot], o_hbm.at[pl.ds(out_base + k * _CHUNK, _CHUNK)],
            out_sem.at[slot])

    for b in range(min(_NBUF, _NCHUNKS)):
        in_copy(b, b).start()
    for k in range(_NCHUNKS):
        slot = k % _NBUF
        in_copy(k, slot).wait()
        out_copy(k, slot).start()
        nk = k + _NBUF
        if nk < _NCHUNKS:
            out_copy(k, slot).wait()
            in_copy(nk, slot).start()
    for k in range(max(0, _NCHUNKS - _NBUF), _NCHUNKS):
        out_copy(k, k % _NBUF).wait()


_sc_copy = pl.kernel(
    _sc_body,
    mesh=plsc.VectorSubcoreMesh(core_axis_name="c", subcore_axis_name="s"),
    out_type=jax.ShapeDtypeStruct((_OUT_ROWS, _W), jnp.float32),
    scratch_types=[
        pltpu.VMEM((_NBUF, _CHUNK, _W), jnp.float32),
        pltpu.SemaphoreType.DMA((_NBUF,)),
        pltpu.SemaphoreType.DMA((_NBUF,)),
    ],
)


def kernel(x):
    xf = x.reshape(14 * 2048, _W)
    return _sc_copy(xf).reshape(8, 2048, 2048)
